# Initial kernel scaffold; baseline (speedup 1.0000x reference)
import jax
import jax.numpy as jnp
from jax.experimental import pallas as pl


def kernel(x0, Wr, br, W1, b1, ln1_s, ln1_b, W2, b2, ln2_s, ln2_b):
    raise NotImplementedError("write your pallas kernel here")


# speedup vs baseline: 1.8326x; 1.8326x over previous
import functools

import jax
import jax.numpy as jnp
from jax import lax
from jax.experimental import pallas as pl
from jax.experimental.pallas import tpu as pltpu
from jax.experimental.pallas import tpu_sc as plsc

S = 2048
D = 1024
E = 8
NODE = 1024
CAP = 512
NTOT = S * 2
EPS = 1e-6
NW = 32
EPW = NTOT // NW
TPW = S // NW
DUMPN = NTOT + 16


def _router_body(x_ref, wr_ref, br_ref, ss_ref, sg_ref, gg_ref):
    x = x_ref[...]
    logits = jnp.dot(x, wr_ref[...], preferred_element_type=jnp.float32)
    logits = logits + br_ref[...]
    eiota = lax.broadcasted_iota(jnp.int32, (S, E), 1)
    m1 = jnp.max(logits, axis=1, keepdims=True)
    e1 = jnp.min(jnp.where(logits == m1, eiota, E), axis=1, keepdims=True)
    l2 = jnp.where(eiota == e1, -jnp.inf, logits)
    m2 = jnp.max(l2, axis=1, keepdims=True)
    e2 = jnp.min(jnp.where(l2 == m2, eiota, E), axis=1, keepdims=True)
    ed = jnp.exp(m2 - m1)
    g1 = 1.0 / (1.0 + ed)
    g2 = ed / (1.0 + ed)
    oh1 = (eiota == e1).astype(jnp.float32)
    oh2 = (eiota == e2).astype(jnp.float32)
    h = oh1 + oh2
    CH = 256
    riota = lax.broadcasted_iota(jnp.int32, (CH, CH), 0)
    ciota = lax.broadcasted_iota(jnp.int32, (CH, CH), 1)
    tri = (ciota < riota).astype(jnp.float32)
    carry = jnp.zeros((1, E), jnp.float32)
    parts = []
    for c in range(S // CH):
        hc = lax.slice_in_dim(h, c * CH, (c + 1) * CH, axis=0)
        parts.append(jnp.dot(tri, hc, preferred_element_type=jnp.float32) + carry)
        carry = carry + jnp.sum(hc, axis=0, keepdims=True)
    excl = jnp.concatenate(parts, axis=0)
    pos1 = jnp.sum(excl * oh1, axis=1, keepdims=True).astype(jnp.int32)
    pos2 = jnp.sum(excl * oh2, axis=1, keepdims=True).astype(jnp.int32)
    v1 = pos1 < CAP
    v2 = pos2 < CAP
    slot1 = e1 * CAP + pos1
    slot2 = e2 * CAP + pos2
    tio = lax.broadcasted_iota(jnp.int32, (S, 1), 0)
    dump1 = NTOT + ((2 * tio) % 16)
    dump2 = NTOT + ((2 * tio + 1) % 16)
    ss_ref[...] = jnp.concatenate(
        [jnp.where(v1, slot1, dump1), jnp.where(v2, slot2, dump2)], axis=1)
    sg_ref[...] = jnp.concatenate(
        [jnp.where(v1, slot1, 0), jnp.where(v2, slot2, 0)], axis=1)
    gg_ref[...] = jnp.concatenate(
        [jnp.where(v1, g1, 0.0), jnp.where(v2, g2, 0.0)], axis=1)


_router = pl.pallas_call(
    _router_body,
    out_shape=[
        jax.ShapeDtypeStruct((S, 2), jnp.int32),
        jax.ShapeDtypeStruct((S, 2), jnp.int32),
        jax.ShapeDtypeStruct((S, 2), jnp.float32),
    ],
)


def _expert_body(x_ref, w1_ref, b1_ref, s1_ref, bb1_ref,
                 w2_ref, b2_ref, s2_ref, bb2_ref, y_ref):
    x = x_ref[...]
    h = jnp.dot(x, w1_ref[0], preferred_element_type=jnp.float32) + b1_ref[0]
    m = jnp.mean(h, axis=1, keepdims=True)
    v = jnp.mean((h - m) * (h - m), axis=1, keepdims=True)
    h = (h - m) / jnp.sqrt(v + EPS) * s1_ref[0] + bb1_ref[0]
    h = jnp.maximum(h, 0.0)
    y = jnp.dot(h, w2_ref[0], preferred_element_type=jnp.float32) + b2_ref[0]
    m2 = jnp.mean(y, axis=1, keepdims=True)
    v2 = jnp.mean((y - m2) * (y - m2), axis=1, keepdims=True)
    y_ref[...] = (y - m2) / jnp.sqrt(v2 + EPS) * s2_ref[0] + bb2_ref[0]


_experts = pl.pallas_call(
    _expert_body,
    grid=(E,),
    in_specs=[
        pl.BlockSpec((CAP, D), lambda e: (e, 0)),
        pl.BlockSpec((1, D, NODE), lambda e: (e, 0, 0)),
        pl.BlockSpec((1, 1, NODE), lambda e: (e, 0, 0)),
        pl.BlockSpec((1, 1, NODE), lambda e: (e, 0, 0)),
        pl.BlockSpec((1, 1, NODE), lambda e: (e, 0, 0)),
        pl.BlockSpec((1, NODE, D), lambda e: (e, 0, 0)),
        pl.BlockSpec((1, 1, D), lambda e: (e, 0, 0)),
        pl.BlockSpec((1, 1, D), lambda e: (e, 0, 0)),
        pl.BlockSpec((1, 1, D), lambda e: (e, 0, 0)),
    ],
    out_specs=pl.BlockSpec((CAP, D), lambda e: (e, 0)),
    out_shape=jax.ShapeDtypeStruct((NTOT, D), jnp.float32),
)


@functools.cache
def _make_dispatch():
    mesh = plsc.VectorSubcoreMesh(core_axis_name="c", subcore_axis_name="s")
    return functools.partial(
        pl.kernel,
        mesh=mesh,
        out_type=jax.ShapeDtypeStruct((NTOT, D), jnp.float32),
        scratch_types=[
            pltpu.VMEM((NTOT,), jnp.int32),
            pltpu.VMEM((DUMPN,), jnp.int32),
            pltpu.VMEM((64, D), jnp.float32),
            pltpu.SemaphoreType.DMA,
        ],
        compiler_params=pltpu.CompilerParams(needs_layout_passes=False),
    )(_dispatch_body)


def _dispatch_body(x_hbm, ss_hbm, out_hbm, sf_v, src_v, rows_v, sem):
    wid = lax.axis_index("s") * 2 + lax.axis_index("c")
    pltpu.sync_copy(ss_hbm, sf_v)

    def zbody(i, c):
        src_v[pl.ds(i * 16, 16)] = jnp.zeros((16,), jnp.int32)
        return c

    lax.fori_loop(0, DUMPN // 16, zbody, 0)

    def sbody(i, c):
        idxv = sf_v[pl.ds(i * 16, 16)]
        ent = lax.iota(jnp.int32, 16) + i * 16
        plsc.store_scatter(src_v, [idxv], lax.shift_right_logical(ent, 1))
        return c

    lax.fori_loop(0, NTOT // 16, sbody, 0)

    for cch in range(2):
        base = wid * EPW + cch * 64
        pltpu.async_copy(x_hbm.at[src_v.at[pl.ds(base, 64)]], rows_v, sem).wait()
        pltpu.sync_copy(rows_v, out_hbm.at[pl.ds(base, 64)])


@functools.cache
def _make_combine():
    mesh = plsc.VectorSubcoreMesh(core_axis_name="c", subcore_axis_name="s")
    return functools.partial(
        pl.kernel,
        mesh=mesh,
        out_type=jax.ShapeDtypeStruct((S, D), jnp.float32),
        scratch_types=[
            pltpu.VMEM((EPW,), jnp.int32),
            pltpu.VMEM((EPW, 16), jnp.float32),
            pltpu.VMEM((32, D), jnp.float32),
            pltpu.VMEM((16, D), jnp.float32),
            pltpu.VMEM((16, D), jnp.float32),
            pltpu.SemaphoreType.DMA,
        ],
        compiler_params=pltpu.CompilerParams(needs_layout_passes=False),
    )(_combine_body)


def _combine_body(x_hbm, y_hbm, sg_hbm, g_hbm, out_hbm,
                  idx_v, gate_v, ybuf, xbuf, obuf, sem):
    wid = lax.axis_index("s") * 2 + lax.axis_index("c")
    ebase = wid * EPW
    pltpu.sync_copy(sg_hbm.at[pl.ds(ebase, EPW)], idx_v)
    pltpu.sync_copy(g_hbm.at[pl.ds(ebase, EPW)], gate_v)
    for cch in range(4):
        pltpu.async_copy(y_hbm.at[idx_v.at[pl.ds(cch * 32, 32)]], ybuf, sem).wait()
        tbase = wid * TPW + cch * 16
        pltpu.sync_copy(x_hbm.at[pl.ds(tbase, 16)], xbuf)
        for j in range(16):
            g0 = gate_v[cch * 32 + 2 * j]
            g1 = gate_v[cch * 32 + 2 * j + 1]

            def cb(k, c, j=j, g0=g0, g1=g1):
                sl = pl.ds(k * 16, 16)
                obuf[j, sl] = jnp.maximum(
                    xbuf[j, sl] + g0 * ybuf[2 * j, sl] + g1 * ybuf[2 * j + 1, sl],
                    0.0)
                return c

            lax.fori_loop(0, D // 16, cb, 0)
        pltpu.sync_copy(obuf, out_hbm.at[pl.ds(tbase, 16)])


def kernel(x0, Wr, br, W1, b1, ln1_s, ln1_b, W2, b2, ln2_s, ln2_b):
    x = x0.reshape(S, D)
    ss, sg, gg = _router(x, Wr, br.reshape(1, E))
    xin = _make_dispatch()(x, ss.reshape(NTOT))
    y = _experts(xin, W1,
                 b1.reshape(E, 1, NODE), ln1_s.reshape(E, 1, NODE),
                 ln1_b.reshape(E, 1, NODE), W2,
                 b2.reshape(E, 1, D), ln2_s.reshape(E, 1, D),
                 ln2_b.reshape(E, 1, D))
    gb = jnp.broadcast_to(gg.reshape(NTOT, 1), (NTOT, 16))
    out = _make_combine()(x, y, sg.reshape(NTOT), gb)
    return out.reshape(x0.shape)
